# baseline (speedup 1.0000x reference)
import functools

import jax
import jax.numpy as jnp
from jax import lax
from jax.experimental import pallas as pl
from jax.experimental.pallas import tpu as pltpu
from jax.experimental.pallas import tpu_sc as plsc

_BM = 1024
_NSUB = 16
_NL = 16


def _matmul_loss_body(f_ref, w_ref, b_ref, t_ref, wt_ref, pred_ref, loss_ref):
    acc = jnp.dot(f_ref[...], w_ref[...], preferred_element_type=jnp.float32)
    acc = acc + b_ref[...]
    pred_ref[...] = acc
    rowmax = jnp.max(acc, axis=1, keepdims=True)
    sumexp = jnp.sum(jnp.exp(acc - rowmax), axis=1, keepdims=True)
    lse = rowmax + jnp.log(sumexp)
    cols = lax.broadcasted_iota(jnp.int32, acc.shape, 1)
    tlogit = jnp.sum(jnp.where(cols == t_ref[...], acc, 0.0), axis=1,
                     keepdims=True)
    loss_ref[...] = (lse - tlogit) * wt_ref[...]


def _dyn_gather(x, idx):
    return lax.gather(
        x, idx[:, None],
        lax.GatherDimensionNumbers(offset_dims=(), collapsed_slice_dims=(0,),
                                   start_index_map=(0,)),
        slice_sizes=(1,),
        mode=lax.GatherScatterMode.PROMISE_IN_BOUNDS)


def _lane_total(x):
    i = lax.iota(jnp.int32, _NL)
    for sh in (1, 2, 4, 8):
        x = x + _dyn_gather(x, i ^ sh)
    return x


def _make_sc_select(m: int, k: int):
    chunk = m // _NSUB
    nv = chunk // _NL
    mesh = plsc.VectorSubcoreMesh(core_axis_name="c", subcore_axis_name="s")

    @functools.partial(
        pl.kernel,
        mesh=mesh,
        out_type=[
            jax.ShapeDtypeStruct((_NL,), jnp.float32),
            jax.ShapeDtypeStruct((_NL,), jnp.int32),
            jax.ShapeDtypeStruct((_NL,), jnp.int32),
        ],
        scratch_types=[
            pltpu.VMEM((chunk,), jnp.float32),
            pltpu.VMEM((chunk,), jnp.int32),
            pltpu.VMEM((_NL,), jnp.int32),
            pltpu.VMEM((_NSUB * _NL,), jnp.int32),
            pltpu.VMEM((_NL,), jnp.float32),
            pltpu.VMEM((_NSUB * _NL,), jnp.float32),
            pltpu.VMEM_SHARED((_NSUB * _NL,), jnp.int32),
            pltpu.VMEM_SHARED((_NSUB * _NL,), jnp.float32),
        ],
    )
    def sel(loss_hbm, lossi_hbm, sum_hbm, cnt_hbm, thr_hbm, x_v, xi_v,
            stage_i, cnts_v, stage_f, sums_v, cnt_sh, sum_sh):
        c = lax.axis_index("c")
        s = lax.axis_index("s")
        one_i = jnp.full((_NL,), 1, jnp.int32)
        zero_i = jnp.full((_NL,), 0, jnp.int32)
        zero_f = jnp.full((_NL,), 0.0, jnp.float32)
        pltpu.sync_copy(loss_hbm.at[pl.ds(s * chunk, chunk)], x_v)
        pltpu.sync_copy(lossi_hbm.at[pl.ds(s * chunk, chunk)], xi_v)

        def count_ge(mid_vec):
            def body(j, acc):
                ge = xi_v[pl.ds(j * _NL, _NL)] >= mid_vec
                return acc + jnp.where(ge, one_i, zero_i)

            return lax.fori_loop(0, nv, body, zero_i)

        def merge_i32(vec):
            stage_i[...] = vec
            pltpu.sync_copy(stage_i, cnt_sh.at[pl.ds(s * _NL, _NL)])
            plsc.subcore_barrier()
            pltpu.sync_copy(cnt_sh, cnts_v)

            def rsum(i, a):
                return a + cnts_v[pl.ds(i * _NL, _NL)]

            tot = lax.fori_loop(0, _NSUB, rsum, zero_i)
            plsc.subcore_barrier()
            return _lane_total(tot)

        k_vec = jnp.full((_NL,), k, jnp.int32)

        def bisect(_, carry):
            lo, hi = carry
            mid = lo + jnp.right_shift(hi - lo + one_i, one_i)
            total = merge_i32(count_ge(mid))
            take = total >= k_vec
            return (jnp.where(take, mid, lo),
                    jnp.where(take, hi, mid - one_i))

        lo, _ = lax.fori_loop(
            0, 31, bisect,
            (zero_i, jnp.full((_NL,), 0x7F800000, jnp.int32)))

        def fbody(j, carry):
            sacc, cacc = carry
            xv = x_v[pl.ds(j * _NL, _NL)]
            gt = xi_v[pl.ds(j * _NL, _NL)] > lo
            return (sacc + jnp.where(gt, xv, zero_f),
                    cacc + jnp.where(gt, one_i, zero_i))

        sacc, cacc = lax.fori_loop(0, nv, fbody, (zero_f, zero_i))

        cnt_gt = merge_i32(cacc)

        stage_f[...] = sacc
        pltpu.sync_copy(stage_f, sum_sh.at[pl.ds(s * _NL, _NL)])
        plsc.subcore_barrier()
        pltpu.sync_copy(sum_sh, sums_v)

        def rsumf(i, a):
            return a + sums_v[pl.ds(i * _NL, _NL)]

        sum_gt = _lane_total(lax.fori_loop(0, _NSUB, rsumf, zero_f))

        @pl.when((c == 0) & (s == 0))
        def _():
            stage_f[...] = sum_gt
            pltpu.sync_copy(stage_f, sum_hbm)
            stage_i[...] = cnt_gt
            pltpu.sync_copy(stage_i, cnt_hbm)
            stage_i[...] = lo
            pltpu.sync_copy(stage_i, thr_hbm)

    return sel


def kernel(features, targets, weights, W, b, interpret=False):
    m, d = features.shape
    n = W.shape[1]
    num_ohem = max(int(m * 0.7), 16)

    pred, losses = pl.pallas_call(
        _matmul_loss_body,
        grid=(m // _BM,),
        in_specs=[
            pl.BlockSpec((_BM, d), lambda i: (i, 0)),
            pl.BlockSpec((d, n), lambda i: (0, 0)),
            pl.BlockSpec((1, n), lambda i: (0, 0)),
            pl.BlockSpec((_BM, 1), lambda i: (i, 0)),
            pl.BlockSpec((_BM, 1), lambda i: (i, 0)),
        ],
        out_specs=[
            pl.BlockSpec((_BM, n), lambda i: (i, 0)),
            pl.BlockSpec((_BM, 1), lambda i: (i, 0)),
        ],
        out_shape=[
            jax.ShapeDtypeStruct((m, n), jnp.float32),
            jax.ShapeDtypeStruct((m, 1), jnp.float32),
        ],
        interpret=interpret,
    )(
        features,
        W,
        b.reshape(1, n),
        targets.astype(jnp.int32).reshape(m, 1),
        weights.reshape(m, 1),
    )

    loss_flat = losses.reshape(m)
    loss_bits = lax.bitcast_convert_type(loss_flat, jnp.int32)
    sel = _make_sc_select(m, num_ohem)
    sum_v, cnt_v, thr_v = sel(loss_flat, loss_bits)

    t = lax.bitcast_convert_type(thr_v[0], jnp.float32)
    final = (sum_v[0] + (num_ohem - cnt_v[0]).astype(jnp.float32) * t) / num_ohem
    return final, pred


# speedup vs baseline: 1.0093x; 1.0093x over previous
import functools

import jax
import jax.numpy as jnp
from jax import lax
from jax.experimental import pallas as pl
from jax.experimental.pallas import tpu as pltpu
from jax.experimental.pallas import tpu_sc as plsc

_BM = 1024
_NSUB = 16
_NL = 16


def _matmul_loss_body(f_ref, w_ref, b_ref, t_ref, wt_ref, pred_ref, loss_ref):
    acc = jnp.dot(f_ref[...], w_ref[...], preferred_element_type=jnp.float32)
    acc = acc + b_ref[...]
    pred_ref[...] = acc
    lse = jnp.log(jnp.sum(jnp.exp(acc), axis=1, keepdims=True))
    cols = lax.broadcasted_iota(jnp.int32, acc.shape, 1)
    tlogit = jnp.sum(jnp.where(cols == t_ref[...], acc, 0.0), axis=1,
                     keepdims=True)
    loss_ref[...] = (lse - tlogit) * wt_ref[...]


def _dyn_gather(x, idx):
    return lax.gather(
        x, idx[:, None],
        lax.GatherDimensionNumbers(offset_dims=(), collapsed_slice_dims=(0,),
                                   start_index_map=(0,)),
        slice_sizes=(1,),
        mode=lax.GatherScatterMode.PROMISE_IN_BOUNDS)


def _lane_total(x):
    i = lax.iota(jnp.int32, _NL)
    for sh in (1, 2, 4, 8):
        x = x + _dyn_gather(x, i ^ sh)
    return x


def _make_sc_select(m: int, k: int):
    chunk = m // _NSUB
    nv = chunk // _NL
    mesh = plsc.VectorSubcoreMesh(core_axis_name="c", subcore_axis_name="s")

    @functools.partial(
        pl.kernel,
        mesh=mesh,
        out_type=[
            jax.ShapeDtypeStruct((_NL,), jnp.float32),
            jax.ShapeDtypeStruct((_NL,), jnp.int32),
            jax.ShapeDtypeStruct((_NL,), jnp.int32),
        ],
        scratch_types=[
            pltpu.VMEM((chunk,), jnp.float32),
            pltpu.VMEM((chunk,), jnp.int32),
            pltpu.VMEM((_NL,), jnp.int32),
            pltpu.VMEM((_NSUB * _NL,), jnp.int32),
            pltpu.VMEM((_NL,), jnp.float32),
            pltpu.VMEM((_NSUB * _NL,), jnp.float32),
            pltpu.VMEM_SHARED((_NSUB * _NL,), jnp.int32),
            pltpu.VMEM_SHARED((_NSUB * _NL,), jnp.float32),
        ],
    )
    def sel(loss_hbm, lossi_hbm, sum_hbm, cnt_hbm, thr_hbm, x_v, xi_v,
            stage_i, cnts_v, stage_f, sums_v, cnt_sh, sum_sh):
        c = lax.axis_index("c")
        s = lax.axis_index("s")
        one_i = jnp.full((_NL,), 1, jnp.int32)
        zero_i = jnp.full((_NL,), 0, jnp.int32)
        zero_f = jnp.full((_NL,), 0.0, jnp.float32)
        pltpu.sync_copy(loss_hbm.at[pl.ds(s * chunk, chunk)], x_v)
        pltpu.sync_copy(lossi_hbm.at[pl.ds(s * chunk, chunk)], xi_v)

        def count_ge(mid_vec):
            def body(j, acc):
                ge = xi_v[pl.ds(j * _NL, _NL)] >= mid_vec
                return acc + jnp.where(ge, one_i, zero_i)

            return lax.fori_loop(0, nv, body, zero_i)

        def merge_i32(vec):
            stage_i[...] = vec
            pltpu.sync_copy(stage_i, cnt_sh.at[pl.ds(s * _NL, _NL)])
            plsc.subcore_barrier()
            pltpu.sync_copy(cnt_sh, cnts_v)

            def rsum(i, a):
                return a + cnts_v[pl.ds(i * _NL, _NL)]

            tot = lax.fori_loop(0, _NSUB, rsum, zero_i)
            plsc.subcore_barrier()
            return _lane_total(tot)

        k_vec = jnp.full((_NL,), k, jnp.int32)

        def bisect(_, carry):
            lo, hi = carry
            mid = lo + jnp.right_shift(hi - lo + one_i, one_i)
            total = merge_i32(count_ge(mid))
            take = total >= k_vec
            return (jnp.where(take, mid, lo),
                    jnp.where(take, hi, mid - one_i))

        lo, _ = lax.fori_loop(
            0, 22, bisect,
            (zero_i, jnp.full((_NL,), 0x7F800000, jnp.int32)))

        def fbody(j, carry):
            sacc, cacc = carry
            xv = x_v[pl.ds(j * _NL, _NL)]
            gt = xi_v[pl.ds(j * _NL, _NL)] > lo
            return (sacc + jnp.where(gt, xv, zero_f),
                    cacc + jnp.where(gt, one_i, zero_i))

        sacc, cacc = lax.fori_loop(0, nv, fbody, (zero_f, zero_i))

        cnt_gt = merge_i32(cacc)

        stage_f[...] = sacc
        pltpu.sync_copy(stage_f, sum_sh.at[pl.ds(s * _NL, _NL)])
        plsc.subcore_barrier()
        pltpu.sync_copy(sum_sh, sums_v)

        def rsumf(i, a):
            return a + sums_v[pl.ds(i * _NL, _NL)]

        sum_gt = _lane_total(lax.fori_loop(0, _NSUB, rsumf, zero_f))

        @pl.when((c == 0) & (s == 0))
        def _():
            stage_f[...] = sum_gt
            pltpu.sync_copy(stage_f, sum_hbm)
            stage_i[...] = cnt_gt
            pltpu.sync_copy(stage_i, cnt_hbm)
            stage_i[...] = lo
            pltpu.sync_copy(stage_i, thr_hbm)

    return sel


def kernel(features, targets, weights, W, b, interpret=False):
    m, d = features.shape
    n = W.shape[1]
    num_ohem = max(int(m * 0.7), 16)

    pred, losses = pl.pallas_call(
        _matmul_loss_body,
        grid=(m // _BM,),
        in_specs=[
            pl.BlockSpec((_BM, d), lambda i: (i, 0)),
            pl.BlockSpec((d, n), lambda i: (0, 0)),
            pl.BlockSpec((1, n), lambda i: (0, 0)),
            pl.BlockSpec((_BM, 1), lambda i: (i, 0)),
            pl.BlockSpec((_BM, 1), lambda i: (i, 0)),
        ],
        out_specs=[
            pl.BlockSpec((_BM, n), lambda i: (i, 0)),
            pl.BlockSpec((_BM, 1), lambda i: (i, 0)),
        ],
        out_shape=[
            jax.ShapeDtypeStruct((m, n), jnp.float32),
            jax.ShapeDtypeStruct((m, 1), jnp.float32),
        ],
        interpret=interpret,
    )(
        features,
        W,
        b.reshape(1, n),
        targets.astype(jnp.int32).reshape(m, 1),
        weights.reshape(m, 1),
    )

    loss_flat = losses.reshape(m)
    loss_bits = lax.bitcast_convert_type(loss_flat, jnp.int32)
    sel = _make_sc_select(m, num_ohem)
    sum_v, cnt_v, thr_v = sel(loss_flat, loss_bits)

    t = lax.bitcast_convert_type(thr_v[0], jnp.float32)
    final = (sum_v[0] + (num_ohem - cnt_v[0]).astype(jnp.float32) * t) / num_ohem
    return final, pred
